# Initial kernel scaffold; baseline (speedup 1.0000x reference)
import jax
import jax.numpy as jnp
from jax.experimental import pallas as pl


def kernel(inputs, embedding):
    raise NotImplementedError("write your pallas kernel here")


# speedup vs baseline: 1.0222x; 1.0222x over previous
import functools

import jax
import jax.numpy as jnp
from jax import lax
from jax.experimental import pallas as pl
from jax.experimental.pallas import tpu as pltpu
from jax.experimental.pallas import tpu_sc as plsc

NUM_ROWS = 1000000
D = 32
B = 16384 * 50
CHUNK = 128
NC = 2
NS = 16
NW = NC * NS
B_PER_W = B // NW
N_CHUNKS = B_PER_W // CHUNK


def _embed_gather(table, idx2d):
    mesh = plsc.VectorSubcoreMesh(core_axis_name="c", subcore_axis_name="s")

    @functools.partial(
        pl.kernel,
        mesh=mesh,
        out_type=jax.ShapeDtypeStruct((B, D), jnp.float32),
        compiler_params=pltpu.CompilerParams(use_tc_tiling_on_sc=False),
        scratch_types=[
            pltpu.VMEM((N_CHUNKS, CHUNK), jnp.int32),
            pltpu.VMEM((CHUNK, D), jnp.float32),
            pltpu.SemaphoreType.DMA,
        ],
    )
    def k(table_hbm, idx_hbm, out_hbm, idx_v, rows_v, sem):
        wid = lax.axis_index("s") * NC + lax.axis_index("c")
        row_base = wid * N_CHUNKS
        pltpu.sync_copy(idx_hbm.at[pl.ds(row_base, N_CHUNKS)], idx_v)

        def body(j, carry):
            pltpu.async_copy(table_hbm.at[idx_v.at[j]], rows_v, sem).wait()
            pltpu.sync_copy(
                rows_v, out_hbm.at[pl.ds((row_base + j) * CHUNK, CHUNK)])
            return carry

        lax.fori_loop(0, N_CHUNKS, body, 0)

    return k(table, idx2d)


def kernel(inputs, embedding):
    idx2d = inputs.reshape(B // CHUNK, CHUNK).astype(jnp.int32)
    out = _embed_gather(embedding, idx2d)
    return out.reshape(inputs.shape + (D,))


# speedup vs baseline: 1.1019x; 1.0780x over previous
import functools

import jax
import jax.numpy as jnp
from jax import lax
from jax.experimental import pallas as pl
from jax.experimental.pallas import tpu as pltpu
from jax.experimental.pallas import tpu_sc as plsc

NUM_ROWS = 1000000
D = 32
B = 16384 * 50
CHUNK = 1024
NC = 2
NS = 16
NW = NC * NS
B_PER_W = B // NW
N_CHUNKS = B_PER_W // CHUNK


def _embed_gather(table, idx2d):
    mesh = plsc.VectorSubcoreMesh(core_axis_name="c", subcore_axis_name="s")

    @functools.partial(
        pl.kernel,
        mesh=mesh,
        out_type=jax.ShapeDtypeStruct((B, D), jnp.float32),
        compiler_params=pltpu.CompilerParams(use_tc_tiling_on_sc=False),
        scratch_types=[
            pltpu.VMEM((N_CHUNKS, CHUNK), jnp.int32),
            pltpu.VMEM((CHUNK, D), jnp.float32),
            pltpu.SemaphoreType.DMA,
        ],
    )
    def k(table_hbm, idx_hbm, out_hbm, idx_v, rows_v, sem):
        wid = lax.axis_index("s") * NC + lax.axis_index("c")
        row_base = wid * N_CHUNKS
        pltpu.sync_copy(idx_hbm.at[pl.ds(row_base, N_CHUNKS)], idx_v)

        def body(j, carry):
            pltpu.async_copy(table_hbm.at[idx_v.at[j]], rows_v, sem).wait()
            pltpu.sync_copy(
                rows_v, out_hbm.at[pl.ds((row_base + j) * CHUNK, CHUNK)])
            return carry

        lax.fori_loop(0, N_CHUNKS, body, 0)

    return k(table, idx2d)


def kernel(inputs, embedding):
    idx2d = inputs.reshape(B // CHUNK, CHUNK).astype(jnp.int32)
    out = _embed_gather(embedding, idx2d)
    return out.reshape(inputs.shape + (D,))


# speedup vs baseline: 1.1088x; 1.0062x over previous
import functools

import jax
import jax.numpy as jnp
from jax import lax
from jax.experimental import pallas as pl
from jax.experimental.pallas import tpu as pltpu
from jax.experimental.pallas import tpu_sc as plsc

NUM_ROWS = 1000000
D = 32
B = 16384 * 50
CHUNK = 1024
NC = 2
NS = 16
NW = NC * NS
B_PER_W = B // NW
N_CHUNKS = B_PER_W // CHUNK


def _embed_gather(table, idx2d):
    mesh = plsc.VectorSubcoreMesh(core_axis_name="c", subcore_axis_name="s")

    @functools.partial(
        pl.kernel,
        mesh=mesh,
        out_type=jax.ShapeDtypeStruct((B, D), jnp.float32),
        compiler_params=pltpu.CompilerParams(use_tc_tiling_on_sc=False),
        scratch_types=[
            pltpu.VMEM((N_CHUNKS, CHUNK), jnp.int32),
            pltpu.VMEM((2, CHUNK, D), jnp.float32),
            pltpu.SemaphoreType.DMA,
            pltpu.SemaphoreType.DMA,
            pltpu.SemaphoreType.DMA,
            pltpu.SemaphoreType.DMA,
        ],
    )
    def k(table_hbm, idx_hbm, out_hbm, idx_v, rows_v, g0, g1, s0, s1):
        wid = lax.axis_index("s") * NC + lax.axis_index("c")
        row_base = wid * N_CHUNKS
        gsem, ssem = (g0, g1), (s0, s1)
        pltpu.sync_copy(idx_hbm.at[pl.ds(row_base, N_CHUNKS)], idx_v)

        gathers = [None] * N_CHUNKS
        stores = [None] * N_CHUNKS
        gathers[0] = pltpu.async_copy(
            table_hbm.at[idx_v.at[0]], rows_v.at[0], gsem[0])
        for j in range(N_CHUNKS):
            b = j % 2
            gathers[j].wait()
            if j >= 1:
                stores[j - 1].wait()
            if j + 1 < N_CHUNKS:
                gathers[j + 1] = pltpu.async_copy(
                    table_hbm.at[idx_v.at[j + 1]], rows_v.at[1 - b],
                    gsem[1 - b])
            stores[j] = pltpu.async_copy(
                rows_v.at[b],
                out_hbm.at[pl.ds((row_base + j) * CHUNK, CHUNK)], ssem[b])
        stores[N_CHUNKS - 1].wait()

    return k(table, idx2d)


def kernel(inputs, embedding):
    idx2d = inputs.reshape(B // CHUNK, CHUNK).astype(jnp.int32)
    out = _embed_gather(embedding, idx2d)
    return out.reshape(inputs.shape + (D,))


# speedup vs baseline: 1.1123x; 1.0032x over previous
import functools

import jax
import jax.numpy as jnp
from jax import lax
from jax.experimental import pallas as pl
from jax.experimental.pallas import tpu as pltpu
from jax.experimental.pallas import tpu_sc as plsc

NUM_ROWS = 1000000
D = 32
B = 16384 * 50
CHUNK = 512
NBUF = 6
NC = 2
NS = 16
NW = NC * NS
B_PER_W = B // NW
N_CHUNKS = B_PER_W // CHUNK


def _embed_gather(table, idx2d):
    mesh = plsc.VectorSubcoreMesh(core_axis_name="c", subcore_axis_name="s")

    @functools.partial(
        pl.kernel,
        mesh=mesh,
        out_type=jax.ShapeDtypeStruct((B, D), jnp.float32),
        compiler_params=pltpu.CompilerParams(use_tc_tiling_on_sc=False),
        scratch_types=[
            pltpu.VMEM((N_CHUNKS, CHUNK), jnp.int32),
            pltpu.VMEM((NBUF, CHUNK, D), jnp.float32),
        ] + [pltpu.SemaphoreType.DMA] * (2 * NBUF),
    )
    def k(table_hbm, idx_hbm, out_hbm, idx_v, rows_v, *sems):
        gsem, ssem = sems[:NBUF], sems[NBUF:]
        wid = lax.axis_index("s") * NC + lax.axis_index("c")
        row_base = wid * N_CHUNKS
        pltpu.sync_copy(idx_hbm.at[pl.ds(row_base, N_CHUNKS)], idx_v)

        def gather(j):
            return pltpu.async_copy(
                table_hbm.at[idx_v.at[j]], rows_v.at[j % NBUF],
                gsem[j % NBUF])

        def store(j):
            return pltpu.async_copy(
                rows_v.at[j % NBUF],
                out_hbm.at[pl.ds((row_base + j) * CHUNK, CHUNK)],
                ssem[j % NBUF])

        gathers = [None] * N_CHUNKS
        stores = [None] * N_CHUNKS
        for j in range(min(NBUF, N_CHUNKS)):
            gathers[j] = gather(j)
        gathers[0].wait()
        stores[0] = store(0)
        for j in range(1, N_CHUNKS):
            stores[j - 1].wait()
            if j - 1 + NBUF < N_CHUNKS:
                gathers[j - 1 + NBUF] = gather(j - 1 + NBUF)
            gathers[j].wait()
            stores[j] = store(j)
        stores[N_CHUNKS - 1].wait()

    return k(table, idx2d)


def kernel(inputs, embedding):
    idx2d = inputs.reshape(B // CHUNK, CHUNK).astype(jnp.int32)
    out = _embed_gather(embedding, idx2d)
    return out.reshape(inputs.shape + (D,))


# speedup vs baseline: 1.5056x; 1.3536x over previous
import functools

import jax
import jax.numpy as jnp
from jax import lax
from jax.experimental import pallas as pl
from jax.experimental.pallas import tpu as pltpu
from jax.experimental.pallas import tpu_sc as plsc

NUM_ROWS = 1000000
D = 32
NB = 16384
NS_SEQ = 50
B = NB * NS_SEQ
CHUNK = 128
N_UNITS = NS_SEQ * (NB // CHUNK)
NC = 2
NSC = 16
NW = NC * NSC
U_PER_W = N_UNITS // NW
NBUF = 4
BT = NB // CHUNK


def _embed_gather(table, idx2d):
    mesh = plsc.VectorSubcoreMesh(core_axis_name="c", subcore_axis_name="s")

    @functools.partial(
        pl.kernel,
        mesh=mesh,
        out_type=jax.ShapeDtypeStruct((NS_SEQ, D // 8, BT, 8, CHUNK),
                                      jnp.float32),
        compiler_params=pltpu.CompilerParams(
            use_tc_tiling_on_sc=False, needs_layout_passes=False),
        scratch_types=[
            pltpu.VMEM((U_PER_W, CHUNK), jnp.int32),
            pltpu.VMEM((NBUF, CHUNK, D), jnp.float32),
            pltpu.VMEM((NBUF, D // 8, 8, CHUNK), jnp.float32),
        ] + [pltpu.SemaphoreType.DMA] * (2 * NBUF),
    )
    def k(table_hbm, idx_hbm, out_hbm, idx_v, rows_v, tiles_v, *sems):
        gsem, ssem = sems[:NBUF], sems[NBUF:]
        wid = lax.axis_index("s") * NC + lax.axis_index("c")
        u_base = wid * U_PER_W
        pltpu.sync_copy(idx_hbm.at[pl.ds(u_base, U_PER_W)], idx_v)

        def gather(j, b):
            return pltpu.async_copy(
                table_hbm.at[idx_v.at[j]], rows_v.at[b], gsem[b])

        def store(j, b):
            u = u_base + j
            return pltpu.async_copy(
                tiles_v.at[b], out_hbm.at[u // BT, :, u % BT], ssem[b])

        def transpose(b):
            g = rows_v.at[b]
            t = tiles_v.at[b]
            for f in range(D):
                for c in range(CHUNK // 16):
                    bi = lax.iota(jnp.int32, 16) + (c * 16)
                    fv = jnp.full((16,), f, jnp.int32)
                    t[f // 8, f % 8, pl.ds(c * 16, 16)] = plsc.load_gather(
                        g, [bi, fv])

        n_grp = U_PER_W // NBUF
        for b in range(NBUF):
            gather(b, b)

        def grp_body(i, carry):
            for b in range(NBUF):
                j = i * NBUF + b

                @pl.when(i > 0)
                def _():
                    pltpu.make_async_copy(
                        tiles_v.at[b], out_hbm.at[0, :, 0], ssem[b]).wait()

                gather_wait = pltpu.make_async_copy(
                    table_hbm.at[idx_v.at[j]], rows_v.at[b], gsem[b])
                gather_wait.wait()
                transpose(b)

                @pl.when(j + NBUF < U_PER_W)
                def _():
                    gather(j + NBUF, b)

                store(j, b)
            return carry

        lax.fori_loop(0, n_grp, grp_body, 0)
        for b in range(NBUF):
            pltpu.make_async_copy(
                tiles_v.at[b],
                out_hbm.at[0, :, 0], ssem[b]).wait()

    return k(table, idx2d)


def kernel(inputs, embedding):
    idx2d = inputs.astype(jnp.int32).T.reshape(N_UNITS, CHUNK)
    out = _embed_gather(embedding, idx2d)
    return out.transpose(2, 4, 0, 1, 3).reshape(NB, NS_SEQ, D)


# speedup vs baseline: 1.8274x; 1.2137x over previous
import functools

import jax
import jax.numpy as jnp
from jax import lax
from jax.experimental import pallas as pl
from jax.experimental.pallas import tpu as pltpu
from jax.experimental.pallas import tpu_sc as plsc

NUM_ROWS = 1000000
D = 32
NB = 16384
NS_SEQ = 50
B = NB * NS_SEQ
CHUNK = 128
N_UNITS = NS_SEQ * (NB // CHUNK)
NC = 2
NSC = 16
NW = NC * NSC
U_PER_W = N_UNITS // NW
NBUF = 4
BT = NB // CHUNK


def _embed_gather(table, idx2d):
    mesh = plsc.VectorSubcoreMesh(core_axis_name="c", subcore_axis_name="s")

    @functools.partial(
        pl.kernel,
        mesh=mesh,
        out_type=jax.ShapeDtypeStruct((NS_SEQ, D // 8, BT, 8, CHUNK),
                                      jnp.float32),
        compiler_params=pltpu.CompilerParams(
            use_tc_tiling_on_sc=False, needs_layout_passes=False),
        scratch_types=[
            pltpu.VMEM((U_PER_W, CHUNK), jnp.int32),
            pltpu.VMEM((NBUF, CHUNK, D), jnp.float32),
            pltpu.VMEM((NBUF, D // 8, 8, CHUNK), jnp.float32),
        ] + [pltpu.SemaphoreType.DMA] * (2 * NBUF),
    )
    def k(table_hbm, idx_hbm, out_hbm, idx_v, rows_v, tiles_v, *sems):
        gsem, ssem = sems[:NBUF], sems[NBUF:]
        wid = lax.axis_index("s") * NC + lax.axis_index("c")
        u_base = wid * U_PER_W
        pltpu.sync_copy(idx_hbm.at[pl.ds(u_base, U_PER_W)], idx_v)

        def gather(j, b):
            return pltpu.async_copy(
                table_hbm.at[idx_v.at[j]], rows_v.at[b], gsem[b])

        def store(j, b):
            u = u_base + j
            return pltpu.async_copy(
                tiles_v.at[b], out_hbm.at[u // BT, :, u % BT], ssem[b])

        def transpose(b):
            g = rows_v.at[b]
            t = tiles_v.at[b]
            lane = lax.iota(jnp.int32, 16)
            fi_vec = lane % 8
            for h in range(2):
                ft_vec = lane // 8 + (2 * h)
                for bi in range(CHUNK):
                    bi_vec = jnp.full((16,), bi, jnp.int32)
                    plsc.store_scatter(
                        t, [ft_vec, fi_vec, bi_vec],
                        g[bi, pl.ds(h * 16, 16)])

        n_grp = U_PER_W // NBUF
        for b in range(NBUF):
            gather(b, b)

        def grp_body(i, carry):
            for b in range(NBUF):
                j = i * NBUF + b

                @pl.when(i > 0)
                def _():
                    pltpu.make_async_copy(
                        tiles_v.at[b], out_hbm.at[0, :, 0], ssem[b]).wait()

                gather_wait = pltpu.make_async_copy(
                    table_hbm.at[idx_v.at[j]], rows_v.at[b], gsem[b])
                gather_wait.wait()
                transpose(b)

                @pl.when(j + NBUF < U_PER_W)
                def _():
                    gather(j + NBUF, b)

                store(j, b)
            return carry

        lax.fori_loop(0, n_grp, grp_body, 0)
        for b in range(NBUF):
            pltpu.make_async_copy(
                tiles_v.at[b],
                out_hbm.at[0, :, 0], ssem[b]).wait()

    return k(table, idx2d)


def kernel(inputs, embedding):
    idx2d = inputs.astype(jnp.int32).T.reshape(N_UNITS, CHUNK)
    out = _embed_gather(embedding, idx2d)
    return out.transpose(2, 4, 0, 1, 3).reshape(NB, NS_SEQ, D)


# speedup vs baseline: 2.1195x; 1.1598x over previous
import functools

import jax
import jax.numpy as jnp
from jax import lax
from jax.experimental import pallas as pl
from jax.experimental.pallas import tpu as pltpu
from jax.experimental.pallas import tpu_sc as plsc

NUM_ROWS = 1000000
D = 32
NB = 16384
NS_SEQ = 50
B = NB * NS_SEQ
CHUNK = 128
N_UNITS = NS_SEQ * (NB // CHUNK)
NC = 2
NSC = 16
NW = NC * NSC
U_PER_W = N_UNITS // NW
NBUF = 4
BT = NB // CHUNK


def _embed_gather(table, idx2d):
    mesh = plsc.VectorSubcoreMesh(core_axis_name="c", subcore_axis_name="s")

    @functools.partial(
        pl.kernel,
        mesh=mesh,
        out_type=jax.ShapeDtypeStruct((NS_SEQ, D // 8, BT, 8, CHUNK),
                                      jnp.float32),
        compiler_params=pltpu.CompilerParams(
            use_tc_tiling_on_sc=False, needs_layout_passes=False),
        scratch_types=[
            pltpu.VMEM((U_PER_W, CHUNK), jnp.int32),
            pltpu.VMEM((NBUF, CHUNK, D), jnp.float32),
            pltpu.VMEM((NBUF, D // 8, 8, CHUNK), jnp.float32),
        ] + [pltpu.SemaphoreType.DMA] * (2 * NBUF),
    )
    def k(table_hbm, idx_hbm, out_hbm, idx_v, rows_v, tiles_v, *sems):
        gsem, ssem = sems[:NBUF], sems[NBUF:]
        wid = lax.axis_index("s") * NC + lax.axis_index("c")
        u_base = wid * U_PER_W
        pltpu.sync_copy(idx_hbm.at[pl.ds(u_base, U_PER_W)], idx_v)

        def gather(j, b):
            return pltpu.async_copy(
                table_hbm.at[idx_v.at[j]], rows_v.at[b], gsem[b])

        def store(j, b):
            u = u_base + j
            return pltpu.async_copy(
                tiles_v.at[b], out_hbm.at[u // BT, :, u % BT], ssem[b])

        def transpose(b):
            g = rows_v.at[b]
            t = tiles_v.at[b]
            lane = lax.iota(jnp.int32, 16)
            fi_vec = lane % 8
            for h in range(2):
                ft_vec = lane // 8 + (2 * h)

                @plsc.parallel_loop(0, CHUNK, 1, unroll=8)
                def _(bi):
                    bi_vec = jnp.full((16,), 0, jnp.int32) + bi
                    plsc.store_scatter(
                        t, [ft_vec, fi_vec, bi_vec],
                        g[bi, pl.ds(h * 16, 16)])

        n_grp = U_PER_W // NBUF
        for b in range(NBUF):
            gather(b, b)

        def grp_body(i, carry):
            for b in range(NBUF):
                j = i * NBUF + b

                @pl.when(i > 0)
                def _():
                    pltpu.make_async_copy(
                        tiles_v.at[b], out_hbm.at[0, :, 0], ssem[b]).wait()

                gather_wait = pltpu.make_async_copy(
                    table_hbm.at[idx_v.at[j]], rows_v.at[b], gsem[b])
                gather_wait.wait()
                transpose(b)

                @pl.when(j + NBUF < U_PER_W)
                def _():
                    gather(j + NBUF, b)

                store(j, b)
            return carry

        lax.fori_loop(0, n_grp, grp_body, 0)
        for b in range(NBUF):
            pltpu.make_async_copy(
                tiles_v.at[b],
                out_hbm.at[0, :, 0], ssem[b]).wait()

    return k(table, idx2d)


def kernel(inputs, embedding):
    idx2d = inputs.astype(jnp.int32).T.reshape(N_UNITS, CHUNK)
    out = _embed_gather(embedding, idx2d)
    return out.transpose(2, 4, 0, 1, 3).reshape(NB, NS_SEQ, D)


# speedup vs baseline: 2.1521x; 1.0154x over previous
import functools

import jax
import jax.numpy as jnp
from jax import lax
from jax.experimental import pallas as pl
from jax.experimental.pallas import tpu as pltpu
from jax.experimental.pallas import tpu_sc as plsc

NUM_ROWS = 1000000
D = 32
NB = 16384
NS_SEQ = 50
B = NB * NS_SEQ
CHUNK = 128
N_UNITS = NS_SEQ * (NB // CHUNK)
NC = 2
NSC = 16
NW = NC * NSC
U_PER_W = N_UNITS // NW
NBUF = 4
BT = NB // CHUNK


def _embed_gather(table, idx2d):
    mesh = plsc.VectorSubcoreMesh(core_axis_name="c", subcore_axis_name="s")

    @functools.partial(
        pl.kernel,
        mesh=mesh,
        out_type=jax.ShapeDtypeStruct((NS_SEQ, D // 8, BT, 8, CHUNK),
                                      jnp.float32),
        compiler_params=pltpu.CompilerParams(
            use_tc_tiling_on_sc=False, needs_layout_passes=False),
        scratch_types=[
            pltpu.VMEM((U_PER_W, CHUNK), jnp.int32),
            pltpu.VMEM((NBUF, CHUNK, D), jnp.float32),
            pltpu.VMEM((NBUF, D // 8, 8, CHUNK), jnp.float32),
        ] + [pltpu.SemaphoreType.DMA] * (2 * NBUF),
    )
    def k(table_hbm, idx_hbm, out_hbm, idx_v, rows_v, tiles_v, *sems):
        gsem, ssem = sems[:NBUF], sems[NBUF:]
        wid = lax.axis_index("s") * NC + lax.axis_index("c")
        u_base = wid * U_PER_W
        pltpu.sync_copy(idx_hbm.at[pl.ds(u_base, U_PER_W)], idx_v)

        def gather(j, b):
            return pltpu.async_copy(
                table_hbm.at[idx_v.at[j]], rows_v.at[b], gsem[b])

        def store(j, b):
            u = u_base + j
            return pltpu.async_copy(
                tiles_v.at[b], out_hbm.at[u // BT, :, u % BT], ssem[b])

        def transpose(b):
            g = rows_v.at[b]
            t = tiles_v.at[b]
            lane = lax.iota(jnp.int32, 16)
            fi_vec = lane % 8
            ft0 = lane // 8
            ft1 = ft0 + 2

            @plsc.parallel_loop(0, CHUNK, 1, unroll=16)
            def _(bi):
                bi_vec = jnp.full((16,), 0, jnp.int32) + bi
                plsc.store_scatter(
                    t, [ft0, fi_vec, bi_vec], g[bi, pl.ds(0, 16)])
                plsc.store_scatter(
                    t, [ft1, fi_vec, bi_vec], g[bi, pl.ds(16, 16)])

        n_grp = U_PER_W // NBUF
        for b in range(NBUF):
            gather(b, b)

        def grp_body(i, carry):
            for b in range(NBUF):
                j = i * NBUF + b

                @pl.when(i > 0)
                def _():
                    pltpu.make_async_copy(
                        tiles_v.at[b], out_hbm.at[0, :, 0], ssem[b]).wait()

                gather_wait = pltpu.make_async_copy(
                    table_hbm.at[idx_v.at[j]], rows_v.at[b], gsem[b])
                gather_wait.wait()
                transpose(b)

                @pl.when(j + NBUF < U_PER_W)
                def _():
                    gather(j + NBUF, b)

                store(j, b)
            return carry

        lax.fori_loop(0, n_grp, grp_body, 0)
        for b in range(NBUF):
            pltpu.make_async_copy(
                tiles_v.at[b],
                out_hbm.at[0, :, 0], ssem[b]).wait()

    return k(table, idx2d)


def kernel(inputs, embedding):
    idx2d = inputs.astype(jnp.int32).T.reshape(N_UNITS, CHUNK)
    out = _embed_gather(embedding, idx2d)
    return out.transpose(2, 4, 0, 1, 3).reshape(NB, NS_SEQ, D)


# speedup vs baseline: 2.8036x; 1.3027x over previous
import functools

import jax
import jax.numpy as jnp
from jax import lax
from jax.experimental import pallas as pl
from jax.experimental.pallas import tpu as pltpu
from jax.experimental.pallas import tpu_sc as plsc

NUM_ROWS = 1000000
D = 32
NB = 16384
NS_SEQ = 50
B = NB * NS_SEQ
CHUNK = 128
N_UNITS = NS_SEQ * (NB // CHUNK)
NC = 2
NSC = 16
NW = NC * NSC
U_PER_W = N_UNITS // NW
NBUF = 4
BT = NB // CHUNK


def _embed_gather(table, idx2d):
    mesh = plsc.VectorSubcoreMesh(core_axis_name="c", subcore_axis_name="s")

    @functools.partial(
        pl.kernel,
        mesh=mesh,
        out_type=jax.ShapeDtypeStruct((NS_SEQ, D // 8, BT, 8, CHUNK),
                                      jnp.float32),
        compiler_params=pltpu.CompilerParams(
            use_tc_tiling_on_sc=False, needs_layout_passes=False),
        scratch_types=[
            pltpu.VMEM((U_PER_W, CHUNK), jnp.int32),
            pltpu.VMEM((NBUF, CHUNK, D), jnp.float32),
            pltpu.VMEM((NBUF, D // 8, 8, CHUNK), jnp.float32),
        ] + [pltpu.SemaphoreType.DMA] * (2 * NBUF),
    )
    def k(table_hbm, idx_hbm, out_hbm, idx_v, rows_v, tiles_v, *sems):
        gsem, ssem = sems[:NBUF], sems[NBUF:]
        wid = lax.axis_index("s") * NC + lax.axis_index("c")
        u_base = wid * U_PER_W
        pltpu.sync_copy(idx_hbm.at[pl.ds(u_base, U_PER_W)], idx_v)

        def gather(j, b):
            return pltpu.async_copy(
                table_hbm.at[idx_v.at[j]], rows_v.at[b], gsem[b])

        def store(j, b):
            u = u_base + j
            return pltpu.async_copy(
                tiles_v.at[b], out_hbm.at[u // BT, :, u % BT], ssem[b])

        def transpose(b):
            g = rows_v.at[b]
            t = tiles_v.at[b]
            lane = lax.iota(jnp.int32, 16)

            @plsc.parallel_loop(0, CHUNK, 16, unroll=2)
            def _(bi0):
                bi_vec = lane + bi0
                for h in range(2):
                    for k in range(16):
                        frot = (lane + k) % 16 + h * 16
                        v = plsc.load_gather(g, [bi_vec, frot])
                        plsc.store_scatter(
                            t, [frot // 8, frot % 8, bi_vec], v)

        n_grp = U_PER_W // NBUF
        for b in range(NBUF):
            gather(b, b)

        def grp_body(i, carry):
            for b in range(NBUF):
                j = i * NBUF + b

                @pl.when(i > 0)
                def _():
                    pltpu.make_async_copy(
                        tiles_v.at[b], out_hbm.at[0, :, 0], ssem[b]).wait()

                gather_wait = pltpu.make_async_copy(
                    table_hbm.at[idx_v.at[j]], rows_v.at[b], gsem[b])
                gather_wait.wait()
                transpose(b)

                @pl.when(j + NBUF < U_PER_W)
                def _():
                    gather(j + NBUF, b)

                store(j, b)
            return carry

        lax.fori_loop(0, n_grp, grp_body, 0)
        for b in range(NBUF):
            pltpu.make_async_copy(
                tiles_v.at[b],
                out_hbm.at[0, :, 0], ssem[b]).wait()

    return k(table, idx2d)


def kernel(inputs, embedding):
    idx2d = inputs.astype(jnp.int32).T.reshape(N_UNITS, CHUNK)
    out = _embed_gather(embedding, idx2d)
    return out.transpose(2, 4, 0, 1, 3).reshape(NB, NS_SEQ, D)


# speedup vs baseline: 2.8477x; 1.0157x over previous
import functools

import jax
import jax.numpy as jnp
from jax import lax
from jax.experimental import pallas as pl
from jax.experimental.pallas import tpu as pltpu
from jax.experimental.pallas import tpu_sc as plsc

NUM_ROWS = 1000000
D = 32
NB = 16384
NS_SEQ = 50
B = NB * NS_SEQ
CHUNK = 128
N_UNITS = NS_SEQ * (NB // CHUNK)
NC = 2
NSC = 16
NW = NC * NSC
U_PER_W = N_UNITS // NW
NBUF = 4
BT = NB // CHUNK


def _embed_gather(table, idx2d):
    mesh = plsc.VectorSubcoreMesh(core_axis_name="c", subcore_axis_name="s")

    @functools.partial(
        pl.kernel,
        mesh=mesh,
        out_type=jax.ShapeDtypeStruct((NS_SEQ, D // 8, BT, 8, CHUNK),
                                      jnp.float32),
        compiler_params=pltpu.CompilerParams(
            use_tc_tiling_on_sc=False, needs_layout_passes=False),
        scratch_types=[
            pltpu.VMEM((U_PER_W, CHUNK), jnp.int32),
            pltpu.VMEM((NBUF, CHUNK, D), jnp.float32),
            pltpu.VMEM((NBUF, D // 8, 8, CHUNK), jnp.float32),
        ] + [pltpu.SemaphoreType.DMA] * (2 * NBUF),
    )
    def k(table_hbm, idx_hbm, out_hbm, idx_v, rows_v, tiles_v, *sems):
        gsem, ssem = sems[:NBUF], sems[NBUF:]
        wid = lax.axis_index("s") * NC + lax.axis_index("c")
        u_base = wid * U_PER_W
        pltpu.sync_copy(idx_hbm.at[pl.ds(u_base, U_PER_W)], idx_v)

        def gather(j, b):
            return pltpu.async_copy(
                table_hbm.at[idx_v.at[j]], rows_v.at[b], gsem[b])

        def store(j, b):
            u = u_base + j
            return pltpu.async_copy(
                tiles_v.at[b], out_hbm.at[u // BT, :, u % BT], ssem[b])

        def transpose(b):
            g = rows_v.at[b]
            t = tiles_v.at[b]
            lane = lax.iota(jnp.int32, 16)

            @plsc.parallel_loop(0, CHUNK, 16, unroll=2)
            def _(bi0):
                bi_vec = lane + bi0
                for h in range(2):
                    for k in range(16):
                        frot = (lane + k) % 16 + h * 16
                        v = plsc.load_gather(g, [bi_vec, frot])
                        plsc.store_scatter(
                            t, [frot // 8, frot % 8, bi_vec], v)

        n_grp = U_PER_W // NBUF
        for b in range(NBUF):
            gather(b, b)

        def grp_body(i, carry):
            for b in range(NBUF):
                j = i * NBUF + b

                @pl.when(i > 0)
                def _():
                    pltpu.make_async_copy(
                        tiles_v.at[b], out_hbm.at[0, :, 0], ssem[b]).wait()

                gather_wait = pltpu.make_async_copy(
                    table_hbm.at[idx_v.at[j]], rows_v.at[b], gsem[b])
                gather_wait.wait()
                transpose(b)

                @pl.when(j + NBUF < U_PER_W)
                def _():
                    gather(j + NBUF, b)

                store(j, b)
            return carry

        lax.fori_loop(0, n_grp, grp_body, 0)
        for b in range(NBUF):
            pltpu.make_async_copy(
                tiles_v.at[b],
                out_hbm.at[0, :, 0], ssem[b]).wait()

    return k(table, idx2d)


def kernel(inputs, embedding):
    idx2d = (inputs.astype(jnp.int32).T * 4).reshape(N_UNITS, CHUNK)
    table = jnp.pad(embedding, ((0, 0), (0, 96))).reshape(4 * NUM_ROWS, D)
    out = _embed_gather(table, idx2d)
    return out.transpose(2, 4, 0, 1, 3).reshape(NB, NS_SEQ, D)


# speedup vs baseline: 5.8583x; 2.0572x over previous
import functools

import jax
import jax.numpy as jnp
from jax import lax
from jax.experimental import pallas as pl
from jax.experimental.pallas import tpu as pltpu
from jax.experimental.pallas import tpu_sc as plsc

NUM_ROWS = 1000000
D = 32
NB = 16384
NS_SEQ = 50
B = NB * NS_SEQ
CHUNK = 128
N_UNITS = NS_SEQ * (NB // CHUNK)
NC = 2
NSC = 16
NW = NC * NSC
U_PER_W = N_UNITS // NW
NBUF = 4
BT = NB // CHUNK


N_RT = NUM_ROWS // CHUNK
RT_PER_W = N_RT // NW
RT_TAIL = N_RT - RT_PER_W * NW
ABUF = 3


def _table_to_rowmajor(emb_t, tail_lin):
    mesh = plsc.VectorSubcoreMesh(core_axis_name="c", subcore_axis_name="s")

    @functools.partial(
        pl.kernel,
        mesh=mesh,
        out_type=jax.ShapeDtypeStruct((NUM_ROWS * D // 1024, 8, 128),
                                      jnp.float32),
        compiler_params=pltpu.CompilerParams(needs_layout_passes=False),
        scratch_types=[
            pltpu.VMEM((ABUF, D, CHUNK), jnp.float32),
            pltpu.VMEM((ABUF, 4, 8, CHUNK), jnp.float32),
        ] + [pltpu.SemaphoreType.DMA] * (2 * ABUF),
    )
    def k(src_hbm, tail_hbm, out_hbm, gbuf, tbuf, *sems):
        gsem, ssem = sems[:ABUF], sems[ABUF:]
        wid = lax.axis_index("s") * NC + lax.axis_index("c")
        c_base = wid * RT_PER_W
        lane = lax.iota(jnp.int32, 16)

        def load(j, b):
            off = pl.multiple_of((c_base + j) * CHUNK, CHUNK)
            return pltpu.async_copy(
                src_hbm.at[:, pl.ds(off, CHUNK)], gbuf.at[b], gsem[b])

        def store(j, b):
            return pltpu.async_copy(
                tbuf.at[b], out_hbm.at[pl.ds((c_base + j) * 4, 4)], ssem[b])

        def transpose(b, n_ri):
            g = gbuf.at[b]
            t = tbuf.at[b]

            @plsc.parallel_loop(0, n_ri, 16, unroll=2)
            def _(ri0):
                ri_vec = lane + ri0
                for h in range(2):
                    for kk in range(16):
                        frot = (lane + kk) % 16 + h * 16
                        v = plsc.load_gather(g, [frot, ri_vec])
                        flat = ri_vec * D + frot
                        plsc.store_scatter(
                            t,
                            [flat >> 10, (flat >> 7) & 7, flat & 127], v)

        for b in range(ABUF):
            load(b, b)
        n_grp = RT_PER_W // ABUF
        n_main = n_grp * ABUF

        def grp_body(i, carry):
            for b in range(ABUF):
                j = i * ABUF + b

                @pl.when(i > 0)
                def _():
                    pltpu.make_async_copy(
                        tbuf.at[b], out_hbm.at[pl.ds(0, 4)], ssem[b]).wait()

                pltpu.make_async_copy(
                    src_hbm.at[:, pl.ds(
                        pl.multiple_of((c_base + j) * CHUNK, CHUNK), CHUNK)],
                    gbuf.at[b], gsem[b]).wait()
                transpose(b, CHUNK)

                @pl.when(j + ABUF < n_main)
                def _():
                    load(j + ABUF, b)

                store(j, b)
            return carry

        lax.fori_loop(0, n_grp, grp_body, 0)
        for b in range(ABUF):
            pltpu.make_async_copy(
                tbuf.at[b], out_hbm.at[pl.ds(0, 4)], ssem[b]).wait()
        for r in range(n_main, RT_PER_W):
            pltpu.sync_copy(
                src_hbm.at[:, pl.ds(
                    pl.multiple_of((c_base + r) * CHUNK, CHUNK), CHUNK)],
                gbuf.at[0])
            transpose(0, CHUNK)
            pltpu.sync_copy(tbuf.at[0],
                            out_hbm.at[pl.ds((c_base + r) * 4, 4)])

        for e in range(RT_TAIL):
            c = NW * RT_PER_W + e

            @pl.when(wid == e)
            def _():
                pltpu.sync_copy(src_hbm.at[:, pl.ds(c * CHUNK, CHUNK)],
                                gbuf.at[0])
                transpose(0, CHUNK)
                pltpu.sync_copy(tbuf.at[0], out_hbm.at[pl.ds(c * 4, 4)])

        @pl.when(wid == RT_TAIL)
        def _():
            pltpu.sync_copy(tail_hbm, tbuf.at[0, pl.ds(0, 2)])
            pltpu.sync_copy(tbuf.at[0, pl.ds(0, 2)],
                            out_hbm.at[pl.ds(N_RT * 4, 2)])

    return k(emb_t, tail_lin)


def _embed_gather(table, idx2d):
    mesh = plsc.VectorSubcoreMesh(core_axis_name="c", subcore_axis_name="s")

    @functools.partial(
        pl.kernel,
        mesh=mesh,
        out_type=jax.ShapeDtypeStruct((NS_SEQ, D // 8, BT, 8, CHUNK),
                                      jnp.float32),
        compiler_params=pltpu.CompilerParams(
            use_tc_tiling_on_sc=False, needs_layout_passes=False),
        scratch_types=[
            pltpu.VMEM((U_PER_W, CHUNK), jnp.int32),
            pltpu.VMEM((NBUF, CHUNK, D), jnp.float32),
            pltpu.VMEM((NBUF, D // 8, 8, CHUNK), jnp.float32),
        ] + [pltpu.SemaphoreType.DMA] * (2 * NBUF),
    )
    def k(table_hbm, idx_hbm, out_hbm, idx_v, rows_v, tiles_v, *sems):
        gsem, ssem = sems[:NBUF], sems[NBUF:]
        wid = lax.axis_index("s") * NC + lax.axis_index("c")
        u_base = wid * U_PER_W
        pltpu.sync_copy(idx_hbm.at[pl.ds(u_base, U_PER_W)], idx_v)

        def gather(j, b):
            return pltpu.async_copy(
                table_hbm.at[idx_v.at[j]], rows_v.at[b], gsem[b])

        def store(j, b):
            u = u_base + j
            return pltpu.async_copy(
                tiles_v.at[b], out_hbm.at[u // BT, :, u % BT], ssem[b])

        def transpose(b):
            g = rows_v.at[b]
            t = tiles_v.at[b]
            lane = lax.iota(jnp.int32, 16)

            @plsc.parallel_loop(0, CHUNK, 16, unroll=2)
            def _(bi0):
                bi_vec = lane + bi0
                for h in range(2):
                    for k in range(16):
                        frot = (lane + k) % 16 + h * 16
                        v = plsc.load_gather(g, [bi_vec, frot])
                        plsc.store_scatter(
                            t, [frot // 8, frot % 8, bi_vec], v)

        n_grp = U_PER_W // NBUF
        for b in range(NBUF):
            gather(b, b)

        def grp_body(i, carry):
            for b in range(NBUF):
                j = i * NBUF + b

                @pl.when(i > 0)
                def _():
                    pltpu.make_async_copy(
                        tiles_v.at[b], out_hbm.at[0, :, 0], ssem[b]).wait()

                gather_wait = pltpu.make_async_copy(
                    table_hbm.at[idx_v.at[j]], rows_v.at[b], gsem[b])
                gather_wait.wait()
                transpose(b)

                @pl.when(j + NBUF < U_PER_W)
                def _():
                    gather(j + NBUF, b)

                store(j, b)
            return carry

        lax.fori_loop(0, n_grp, grp_body, 0)
        for b in range(NBUF):
            pltpu.make_async_copy(
                tiles_v.at[b],
                out_hbm.at[0, :, 0], ssem[b]).wait()

    return k(table, idx2d)


def kernel(inputs, embedding):
    idx2d = inputs.astype(jnp.int32).T.reshape(N_UNITS, CHUNK)
    tail_lin = embedding[N_RT * CHUNK:].reshape(2, 8, CHUNK)
    table = _table_to_rowmajor(embedding.T, tail_lin).reshape(NUM_ROWS, D)
    out = _embed_gather(table, idx2d)
    return out.transpose(2, 4, 0, 1, 3).reshape(NB, NS_SEQ, D)
